# baseline (speedup 1.0000x reference)
import jax
import jax.numpy as jnp
from jax.experimental import pallas as pl
from jax.experimental.pallas import tpu as pltpu

_ROWS = 1024
_COLS = 27278
_SPLIT = 4
_BLOCK_ROWS = 16
_REGION_ROWS = _ROWS // _SPLIT
_STEPS = _REGION_ROWS // _BLOCK_ROWS


def _masked_mse_body(*refs):
    o_refs = refs[:_SPLIT]
    t_refs = refs[_SPLIT:2 * _SPLIT]
    res_ref = refs[2 * _SPLIT]
    acc_ref, cnt_ref = refs[2 * _SPLIT + 1:]
    i = pl.program_id(0)

    @pl.when(i == 0)
    def _init():
        acc_ref[0] = 0.0
        cnt_ref[0] = 0.0

    s = 0.0
    c = 0.0
    for k in range(_SPLIT):
        o = o_refs[k][...]
        t = t_refs[k][...]
        m = t != -1.0
        d = o - t
        s += jnp.sum(jnp.where(m, d * d, 0.0))
        c += jnp.sum(m.astype(jnp.float32))
    acc_ref[0] += s
    cnt_ref[0] += c

    @pl.when(i == pl.num_programs(0) - 1)
    def _fin():
        res_ref[0, 0] = acc_ref[0] / cnt_ref[0]


def kernel(output, target):
    def spec(k):
        blocks_per_region = _REGION_ROWS // _BLOCK_ROWS
        return pl.BlockSpec(
            (_BLOCK_ROWS, _COLS),
            lambda i, k=k, b=blocks_per_region: (k * b + i, 0),
        )

    specs = [spec(k) for k in range(_SPLIT)]
    res = pl.pallas_call(
        _masked_mse_body,
        grid=(_STEPS,),
        in_specs=specs + specs,
        out_specs=pl.BlockSpec(memory_space=pltpu.SMEM),
        out_shape=jax.ShapeDtypeStruct((1, 1), jnp.float32),
        scratch_shapes=[
            pltpu.SMEM((1,), jnp.float32),
            pltpu.SMEM((1,), jnp.float32),
        ],
    )(*([output] * _SPLIT + [target] * _SPLIT))
    return res.reshape(())
